# baseline (speedup 1.0000x reference)
import functools

import jax
import jax.numpy as jnp
from jax import lax
from jax.experimental import pallas as pl
from jax.experimental.pallas import tpu as pltpu
from jax.experimental.pallas import tpu_sc as plsc

_M, _K, _D = 4, 1024, 256


_BT = 1024


def _argmax_body(x_ref, raw_ref, flatt_ref):
    cols_raw = []
    cols_flat = []
    ngrp = _K // 128
    for m in range(_M):
        run_v = x_ref[0, :, m * _K:m * _K + 128]
        run_g = jnp.zeros((_BT, 128), jnp.int32)
        for g in range(1, ngrp):
            v = x_ref[0, :, m * _K + g * 128:m * _K + (g + 1) * 128]
            upd = v > run_v
            run_g = jnp.where(upd, g, run_g)
            run_v = jnp.maximum(run_v, v)
        mx = jnp.max(run_v, axis=1, keepdims=True)
        lane = lax.broadcasted_iota(jnp.int32, (_BT, 128), 1)
        k_cand = run_g * 128 + lane
        idx = jnp.min(jnp.where(run_v == mx, k_cand, _K), axis=1, keepdims=True)
        cols_raw.append(idx)
        cols_flat.append(idx + m * _K)
    raw_ref[0, :, :] = jnp.concatenate(cols_raw, axis=1)
    flatt_ref[...] = jnp.concatenate(cols_flat, axis=1).T.reshape(
        _M, _BT // _CHUNK, _CHUNK)


def _argmax_call(logits):
    bsz, t, _ = logits.shape
    nt = t // _BT
    return pl.pallas_call(
        _argmax_body,
        grid=(bsz, nt),
        in_specs=[pl.BlockSpec((1, _BT, _M * _K), lambda b, i: (b, i, 0))],
        out_specs=[
            pl.BlockSpec((1, _BT, _M), lambda b, i: (b, i, 0)),
            pl.BlockSpec((_M, _BT // _CHUNK, _CHUNK),
                         lambda b, i, _nt=nt: (0, b * _nt + i, 0)),
        ],
        out_shape=[
            jax.ShapeDtypeStruct((bsz, t, _M), jnp.int32),
            jax.ShapeDtypeStruct((_M, bsz * t // _CHUNK, _CHUNK), jnp.int32),
        ],
    )(logits)


_CHUNK = 128


def _make_sc_gather(bsz, t):
    info = plsc.get_sparse_core_info()
    n_tok = bsz * t
    n_chunks = t // _CHUNK
    mesh = plsc.VectorSubcoreMesh(core_axis_name="c", subcore_axis_name="s")

    @functools.partial(
        pl.kernel,
        mesh=mesh,
        out_type=[
            jax.ShapeDtypeStruct((n_tok, _M * _D), jnp.float32),
            jax.ShapeDtypeStruct((n_tok, _M, _D), jnp.float32),
        ],
        scratch_types=[
            pltpu.VMEM((n_chunks, _CHUNK), jnp.int32),
            pltpu.VMEM((_CHUNK, _D), jnp.float32),
            pltpu.VMEM((_CHUNK, _D), jnp.float32),
            pltpu.VMEM((_CHUNK, _D), jnp.float32),
            pltpu.SemaphoreType.DMA,
            pltpu.SemaphoreType.DMA,
            pltpu.SemaphoreType.DMA,
            pltpu.SemaphoreType.DMA,
            pltpu.SemaphoreType.DMA,
            pltpu.SemaphoreType.DMA,
        ],
    )
    def sc_gather(idx_hbm, table_hbm, out1_hbm, out2_hbm,
                  idx_v, buf0, buf1, buf2,
                  gs0, gs1, gs2, ws0, ws1, ws2):
        wid = lax.axis_index("s") * info.num_cores + lax.axis_index("c")
        b = wid // _M
        m = wid % _M
        base = b * t
        nb = 3
        pltpu.sync_copy(idx_hbm.at[m, pl.ds(b * n_chunks, n_chunks)], idx_v)
        bufs = (buf0, buf1, buf2)
        gsems = (gs0, gs1, gs2)
        wsems = (ws0, ws1, ws2)
        gathers = [None] * nb
        writes = [None] * n_chunks
        for c in range(min(2, n_chunks)):
            gathers[c % nb] = pltpu.async_copy(
                table_hbm.at[idx_v.at[c]], bufs[c % nb], gsems[c % nb])
        for c in range(n_chunks):
            if c + 2 < n_chunks:
                if c - 1 >= 0:
                    w1, w2 = writes[c - 1]
                    w1.wait()
                    w2.wait()
                gathers[(c + 2) % nb] = pltpu.async_copy(
                    table_hbm.at[idx_v.at[c + 2]], bufs[(c + 2) % nb],
                    gsems[(c + 2) % nb])
            gathers[c % nb].wait()
            r0 = base + c * _CHUNK
            w1 = pltpu.async_copy(
                bufs[c % nb],
                out1_hbm.at[pl.ds(r0, _CHUNK), pl.ds(m * _D, _D)],
                wsems[c % nb])
            w2 = pltpu.async_copy(
                bufs[c % nb], out2_hbm.at[pl.ds(r0, _CHUNK), m],
                wsems[c % nb])
            writes[c] = (w1, w2)
        for c in range(max(0, n_chunks - 3), n_chunks):
            if writes[c] is not None:
                w1, w2 = writes[c]
                w1.wait()
                w2.wait()

    return sc_gather, n_chunks


def kernel(logits, embeddings):
    bsz, t, _ = logits.shape
    raw, idx3 = _argmax_call(logits)
    sc_gather, n_chunks = _make_sc_gather(bsz, t)
    table = embeddings.reshape(_M * _K, _D)
    q1, q2 = sc_gather(idx3, table)
    quantized = q1.reshape(bsz, t, _M * _D)
    quantized_stack = q2.reshape(bsz, t, _M, _D)
    return quantized, quantized_stack, raw
